# baseline (speedup 1.0000x reference)
import functools

import jax
import jax.numpy as jnp
from jax import lax
from jax.experimental import pallas as pl
from jax.experimental.pallas import tpu as pltpu
from jax.experimental.pallas import tpu_sc as plsc

D_MODEL = 2048
N_EXP = 16
N_TOKENS = 16384
TILE = 2048
GRID = N_TOKENS // TILE

NC = 2
NS = 16
NW = NC * NS
TPW = N_TOKENS // NW


def _logits_body(x_ref, w_ref, b_ref, logits_ref, imp_ref):
    step = pl.program_id(0)
    l = jnp.dot(x_ref[...], w_ref[...], preferred_element_type=jnp.float32)
    l = l + b_ref[...]
    logits_ref[...] = l
    m1 = jnp.max(l, axis=1, keepdims=True)
    e = jnp.exp(l - m1)
    s = jnp.sum(e, axis=1, keepdims=True)
    imp_part = jnp.sum(e * (1.0 / s), axis=0)

    @pl.when(step == 0)
    def _init():
        imp_ref[...] = jnp.zeros_like(imp_ref)

    imp_ref[...] += imp_part[None, :]


def _route_body(logits_hbm, idx_hbm, wgt_hbm, cnt_hbm, lbuf, idxbuf, wgtbuf,
                cntbuf):
    wid = lax.axis_index("s") * NC + lax.axis_index("c")
    base = wid * TPW
    pltpu.sync_copy(logits_hbm.at[pl.ds(base, TPW), :], lbuf)

    iota = lax.broadcasted_iota(jnp.int32, (N_EXP,), 0)
    neg = jnp.full((N_EXP,), -jnp.inf, jnp.float32)

    def g16(v, idx):
        return lax.gather(
            v, idx[:, None],
            dimension_numbers=lax.GatherDimensionNumbers(
                offset_dims=(), collapsed_slice_dims=(0,),
                start_index_map=(0,)),
            slice_sizes=(1,),
            mode=lax.GatherScatterMode.PROMISE_IN_BOUNDS)

    def top1(v):
        val, idx = v, iota
        for k in (1, 2, 4, 8):
            p = jnp.bitwise_xor(iota, k)
            oval = g16(val, p)
            oidx = g16(idx, p)
            take = (oval > val) | ((oval == val) & (oidx < idx))
            val = jnp.where(take, oval, val)
            idx = jnp.where(take, oidx, idx)
        return val, idx

    def blk_fn(blk, cnt):
        def row_fn(j, carry):
            cnt, accw, acci = carry
            row = lbuf[blk * 8 + j]
            m1, i1 = top1(row)
            masked = jnp.where(iota == i1, neg, row)
            m2, i2 = top1(masked)
            tv = jnp.exp(m2 - m1)
            w1v = 1.0 / (1.0 + tv)
            w2v = tv / (1.0 + tv)
            cnt = (cnt + jnp.where(iota == i1, 1.0, 0.0)
                   + jnp.where(iota == i2, 1.0, 0.0))
            accw = jnp.where(iota == 2 * j, w1v, accw)
            accw = jnp.where(iota == 2 * j + 1, w2v, accw)
            acci = jnp.where(iota == 2 * j, i1, acci)
            acci = jnp.where(iota == 2 * j + 1, i2, acci)
            return cnt, accw, acci

        cnt, accw, acci = lax.fori_loop(
            0, 8, row_fn,
            (cnt, jnp.zeros((N_EXP,), jnp.float32),
             jnp.zeros((N_EXP,), jnp.int32)))
        wgtbuf[pl.ds(blk * 16, 16)] = accw
        idxbuf[pl.ds(blk * 16, 16)] = acci
        return cnt

    cnt = lax.fori_loop(0, TPW // 8, blk_fn, jnp.zeros((N_EXP,), jnp.float32))
    cntbuf[...] = cnt
    pltpu.sync_copy(idxbuf, idx_hbm.at[pl.ds(base * 2, TPW * 2)])
    pltpu.sync_copy(wgtbuf, wgt_hbm.at[pl.ds(base * 2, TPW * 2)])
    pltpu.sync_copy(cntbuf, cnt_hbm.at[wid])


def _loss_body(cnt_ref, imp_ref, loss_ref):
    cnt = jnp.sum(cnt_ref[...], axis=0, keepdims=True)
    load = cnt / float(N_TOKENS * 2)
    importance = imp_ref[...] / float(N_TOKENS)
    loss_ref[...] = (float(N_EXP) * jnp.sum(load * importance)).reshape(1, 1)


def kernel(x, W, b):
    x_flat = x.reshape(N_TOKENS, D_MODEL)
    b2 = b.reshape(1, N_EXP)

    logits, imp = pl.pallas_call(
        _logits_body,
        grid=(GRID,),
        in_specs=[
            pl.BlockSpec((TILE, D_MODEL), lambda i: (i, 0)),
            pl.BlockSpec((D_MODEL, N_EXP), lambda i: (0, 0)),
            pl.BlockSpec((1, N_EXP), lambda i: (0, 0)),
        ],
        out_specs=(
            pl.BlockSpec((TILE, N_EXP), lambda i: (i, 0)),
            pl.BlockSpec((1, N_EXP), lambda i: (0, 0)),
        ),
        out_shape=(
            jax.ShapeDtypeStruct((N_TOKENS, N_EXP), jnp.float32),
            jax.ShapeDtypeStruct((1, N_EXP), jnp.float32),
        ),
    )(x_flat, W, b2)

    mesh = plsc.VectorSubcoreMesh(core_axis_name="c", subcore_axis_name="s")
    route = pl.kernel(
        _route_body,
        out_type=(
            jax.ShapeDtypeStruct((N_TOKENS * 2,), jnp.int32),
            jax.ShapeDtypeStruct((N_TOKENS * 2,), jnp.float32),
            jax.ShapeDtypeStruct((NW, N_EXP), jnp.float32),
        ),
        mesh=mesh,
        scratch_types=[
            pltpu.VMEM((TPW, N_EXP), jnp.float32),
            pltpu.VMEM((TPW * 2,), jnp.int32),
            pltpu.VMEM((TPW * 2,), jnp.float32),
            pltpu.VMEM((N_EXP,), jnp.float32),
        ],
    )
    idx_flat, wgt_flat, counts = route(logits)

    loss = pl.pallas_call(
        _loss_body,
        out_shape=jax.ShapeDtypeStruct((1, 1), jnp.float32),
    )(counts, imp)

    idx = idx_flat.reshape(N_TOKENS, 2)
    wgt = wgt_flat.reshape(N_TOKENS, 2)
    return (idx, wgt, loss.reshape(()), logits)


# speedup vs baseline: 1.5377x; 1.5377x over previous
import functools

import jax
import jax.numpy as jnp
from jax import lax
from jax.experimental import pallas as pl

D_MODEL = 2048
N_EXP = 16
N_TOKENS = 16384
TILE = 2048
GRID = N_TOKENS // TILE


def _router_body(x_ref, w_ref, b_ref,
                 logits_ref, idx_ref, wgt_ref, imp_ref, cnt_ref, loss_ref):
    step = pl.program_id(0)

    l = jnp.dot(x_ref[...], w_ref[...], preferred_element_type=jnp.float32)
    l = l + b_ref[...]
    logits_ref[...] = l

    m1 = jnp.max(l, axis=1, keepdims=True)
    e = jnp.exp(l - m1)
    s = jnp.sum(e, axis=1, keepdims=True)
    imp_part = jnp.sum(e * (1.0 / s), axis=0)

    iota = lax.broadcasted_iota(jnp.int32, (TILE, N_EXP), 1)
    big = jnp.int32(N_EXP)
    eq1 = l == m1
    i1 = jnp.min(jnp.where(eq1, iota, big), axis=1)
    mask1 = iota == i1[:, None]
    l2 = jnp.where(mask1, -jnp.inf, l)
    m2 = jnp.max(l2, axis=1, keepdims=True)
    i2 = jnp.min(jnp.where(l2 == m2, iota, big), axis=1)
    mask2 = iota == i2[:, None]

    t = jnp.exp(m2 - m1)
    denom = 1.0 + t
    w1 = 1.0 / denom
    w2 = t / denom

    idx_ref[...] = jnp.concatenate([i1[:, None], i2[:, None]], axis=1)
    wgt_ref[...] = jnp.concatenate([w1, w2], axis=1)

    cnt_part = jnp.sum(mask1.astype(jnp.float32) + mask2.astype(jnp.float32),
                       axis=0)

    @pl.when(step == 0)
    def _init():
        imp_ref[...] = jnp.zeros_like(imp_ref)
        cnt_ref[...] = jnp.zeros_like(cnt_ref)

    imp_ref[...] += imp_part[None, :]
    cnt_ref[...] += cnt_part[None, :]

    @pl.when(step == GRID - 1)
    def _fin():
        load = cnt_ref[...] / float(N_TOKENS * 2)
        imp = imp_ref[...] / float(N_TOKENS)
        loss_ref[...] = (float(N_EXP) * jnp.sum(load * imp)).reshape(1, 1)


@functools.partial(jax.jit, static_argnames=())
def kernel(x, W, b):
    x_flat = x.reshape(N_TOKENS, D_MODEL)
    b2 = b.reshape(1, N_EXP)

    out_shapes = (
        jax.ShapeDtypeStruct((N_TOKENS, N_EXP), jnp.float32),
        jax.ShapeDtypeStruct((N_TOKENS, 2), jnp.int32),
        jax.ShapeDtypeStruct((N_TOKENS, 2), jnp.float32),
        jax.ShapeDtypeStruct((1, N_EXP), jnp.float32),
        jax.ShapeDtypeStruct((1, N_EXP), jnp.float32),
        jax.ShapeDtypeStruct((1, 1), jnp.float32),
    )
    grid_spec = pl.GridSpec(
        grid=(GRID,),
        in_specs=[
            pl.BlockSpec((TILE, D_MODEL), lambda i: (i, 0)),
            pl.BlockSpec((D_MODEL, N_EXP), lambda i: (0, 0)),
            pl.BlockSpec((1, N_EXP), lambda i: (0, 0)),
        ],
        out_specs=(
            pl.BlockSpec((TILE, N_EXP), lambda i: (i, 0)),
            pl.BlockSpec((TILE, 2), lambda i: (i, 0)),
            pl.BlockSpec((TILE, 2), lambda i: (i, 0)),
            pl.BlockSpec((1, N_EXP), lambda i: (0, 0)),
            pl.BlockSpec((1, N_EXP), lambda i: (0, 0)),
            pl.BlockSpec((1, 1), lambda i: (0, 0)),
        ),
    )
    logits, idx, wgt, _imp, _cnt, loss = pl.pallas_call(
        _router_body,
        grid_spec=grid_spec,
        out_shape=out_shapes,
    )(x_flat, W, b2)
    return (idx, wgt, loss.reshape(()), logits)
